# baseline (speedup 1.0000x reference)
import functools
import math

import jax
import jax.numpy as jnp
from jax import lax
from jax.experimental import pallas as pl
from jax.experimental.pallas import tpu as pltpu, tpu_sc as plsc

_C = 160
_G = _C // 16
_HID = 128
_HG = _HID // 16
_NW = 32


@functools.partial(jax.jit, static_argnums=(2, 3))
def _run(tree_flat, conn_flat, node_num, total_rows):
    num_chunks = total_rows // _C
    min_t = num_chunks // _NW
    max_t = -(-num_chunks // _NW)
    inv_s = 1.0 / math.sqrt(_HID)

    mesh = plsc.VectorSubcoreMesh(core_axis_name="c", subcore_axis_name="s")

    @functools.partial(
        pl.kernel,
        out_type=jax.ShapeDtypeStruct((total_rows, _HID), jnp.float32),
        mesh=mesh,
        scratch_types=[
            pltpu.VMEM((_C,), jnp.int32),
            pltpu.VMEM((_C,), jnp.int32),
            pltpu.VMEM((_C, _HID), jnp.float32),
            pltpu.VMEM((_C, _HID), jnp.float32),
            pltpu.VMEM((_C, _HID), jnp.float32),
            pltpu.VMEM((_C, _HID), jnp.float32),
            pltpu.SemaphoreType.DMA,
            pltpu.SemaphoreType.DMA,
            pltpu.SemaphoreType.DMA,
            pltpu.SemaphoreType.DMA,
        ],
    )
    def k(tree_hbm, conn_hbm, out_hbm, idx_a, idx_b, x_a, x_b, p_a, p_b,
          sem_a, sem_b, semo_a, semo_b):
        wid = lax.axis_index("s") * 2 + lax.axis_index("c")
        n_iter = (num_chunks - 1 - wid) // _NW + 1
        bufs = [(idx_a, x_a, p_a, sem_a, semo_a),
                (idx_b, x_b, p_b, sem_b, semo_b)]

        def fire_loads(t):
            idx_v, x_v, p_v, sem, _ = bufs[t % 2]
            cid = wid + t * _NW
            base = cid * _C
            cps = [pltpu.async_copy(tree_hbm.at[pl.ds(base, _C)], x_v, sem)]
            pltpu.sync_copy(conn_hbm.at[pl.ds(base, _C)], idx_v)
            for j in range(_G):
                batch_base = ((base + j * 16) // node_num) * node_num
                flat_idx = idx_v[pl.ds(j * 16, 16)] + batch_base
                cps.append(pltpu.async_copy(
                    tree_hbm.at[flat_idx], p_v.at[pl.ds(j * 16, 16)], sem))
            return cps

        def compute_store(t):
            _, x_v, p_v, _, semo = bufs[t % 2]
            cid = wid + t * _NW

            @plsc.parallel_loop(0, _C, unroll=1)
            def _row(r):
                xs = []
                ss = []
                ms = []
                for c in range(_HG):
                    xc = x_v[r, pl.ds(c * 16, 16)]
                    sc = p_v[r, pl.ds(c * 16, 16)] - xc
                    xs.append(xc)
                    ss.append(sc)
                    ms.append(sc * xc)
                t0 = [ms[0] + ms[1], ms[2] + ms[3], ms[4] + ms[5], ms[6] + ms[7]]
                t1 = [t0[0] + t0[1], t0[2] + t0[3]]
                acc = t1[0] + t1[1]
                lane = lax.iota(jnp.int32, 16)
                d = acc
                for kk in (8, 4, 2, 1):
                    d = d + d.at[lane ^ kk].get(mode="promise_in_bounds")
                w = 1.0 / (1.0 + jnp.exp(d * (-inv_s)))
                for c in range(_HG):
                    p_v[r, pl.ds(c * 16, 16)] = xs[c] + w * ss[c]

            return pltpu.async_copy(p_v, out_hbm.at[pl.ds(cid * _C, _C)], semo)

        def step(t, loads_t, prev_out):
            if prev_out is not None:
                prev_out.wait()
            next_loads = None
            if t + 1 < min_t:
                next_loads = fire_loads(t + 1)
            for cp in loads_t:
                cp.wait()
            out = compute_store(t)
            return next_loads, out

        loads = fire_loads(0)
        prev_out = None
        outs = []
        for t in range(min_t):
            loads, out = step(t, loads, prev_out)
            prev_out = out
            outs.append(out)
        for t in range(min_t, max_t):
            @pl.when(t < n_iter)
            def _tail():
                for cp in fire_loads(t):
                    cp.wait()
                compute_store(t).wait()
        outs[-1].wait()

    return k(tree_flat, conn_flat)


def kernel(tree_embedding, node_connection, node_mask):
    batch, node_num, hid = tree_embedding.shape
    assert hid == _HID and node_num % 16 == 0
    total_rows = batch * node_num
    assert total_rows % _C == 0
    tree_flat = tree_embedding.reshape(total_rows, hid)
    conn_flat = node_connection.astype(jnp.int32).reshape(total_rows)
    out = _run(tree_flat, conn_flat, node_num, total_rows)
    return out.reshape(batch, node_num, hid)
